# baseline (speedup 1.0000x reference)
import functools

import jax
import jax.numpy as jnp
from jax import lax
from jax.experimental import pallas as pl
from jax.experimental.pallas import tpu as pltpu
from jax.experimental.pallas import tpu_sc as plsc

_ROWS = 8192
_D = 1024
_NC = 2
_RPC = _ROWS // _NC
_CHUNK = 512
_NCHUNK = _RPC // _CHUNK
_NBUF = 3

_mesh = plsc.ScalarSubcoreMesh(axis_name="c", num_cores=_NC)


@functools.partial(
    pl.kernel,
    out_type=jax.ShapeDtypeStruct((_ROWS, _D), jnp.float32),
    mesh=_mesh,
    scratch_types=[
        pltpu.VMEM_SHARED((_NBUF, _CHUNK, _D), jnp.float32),
    ] + [pltpu.SemaphoreType.DMA] * (2 * _NBUF),
)
def _pe_copy(table_hbm, out_hbm, buf, *sems):
    sins = sems[:_NBUF]
    souts = sems[_NBUF:]
    base = lax.axis_index("c") * _RPC
    in_copies = [None] * _NBUF
    out_copies = [None] * _NBUF

    for i in range(min(_NBUF, _NCHUNK)):
        in_copies[i] = pltpu.async_copy(
            table_hbm.at[pl.ds(base + i * _CHUNK, _CHUNK)],
            buf.at[i], sins[i])
    for i in range(_NCHUNK):
        b = i % _NBUF
        in_copies[b].wait()
        out_copies[b] = pltpu.async_copy(
            buf.at[b], out_hbm.at[pl.ds(base + i * _CHUNK, _CHUNK)], souts[b])
        j = i + _NBUF
        if j < _NCHUNK:
            out_copies[b].wait()
            out_copies[b] = None
            in_copies[b] = pltpu.async_copy(
                table_hbm.at[pl.ds(base + j * _CHUNK, _CHUNK)],
                buf.at[b], sins[b])
    for b in range(_NBUF):
        if out_copies[b] is not None:
            out_copies[b].wait()


def kernel(x, pe_weight):
    del x
    return _pe_copy(pe_weight)[None]


# speedup vs baseline: 1.0402x; 1.0402x over previous
import functools

import jax
import jax.numpy as jnp
from jax import lax
from jax.experimental import pallas as pl
from jax.experimental.pallas import tpu as pltpu
from jax.experimental.pallas import tpu_sc as plsc

_ROWS = 8192
_D = 1024
_NC = 2
_NS = 16
_NW = _NC * _NS
_RPW = _ROWS // _NW
_CHUNK = 32
_NCHUNK = _RPW // _CHUNK
_NBUF = 3
_LAG = 1

_mesh = plsc.VectorSubcoreMesh(core_axis_name="c", subcore_axis_name="s")


@functools.partial(
    pl.kernel,
    out_type=jax.ShapeDtypeStruct((_ROWS, _D), jnp.float32),
    mesh=_mesh,
    scratch_types=[
        pltpu.VMEM((_NBUF, _CHUNK, _D), jnp.float32),
    ] + [pltpu.SemaphoreType.DMA] * (2 * _NBUF),
)
def _pe_copy(table_hbm, out_hbm, buf, *sems):
    sins = sems[:_NBUF]
    souts = sems[_NBUF:]
    wid = lax.axis_index("s") * _NC + lax.axis_index("c")
    base = wid * _RPW
    in_copies = [None] * _NBUF
    out_copies = [None] * _NBUF

    for i in range(_NBUF - _LAG):
        in_copies[i] = pltpu.async_copy(
            table_hbm.at[pl.ds(base + i * _CHUNK, _CHUNK)],
            buf.at[i], sins[i])
    for i in range(_NCHUNK):
        b = i % _NBUF
        in_copies[b].wait()
        out_copies[b] = pltpu.async_copy(
            buf.at[b], out_hbm.at[pl.ds(base + i * _CHUNK, _CHUNK)], souts[b])
        j = i + _NBUF - _LAG
        if j < _NCHUNK:
            jb = j % _NBUF
            if out_copies[jb] is not None:
                out_copies[jb].wait()
                out_copies[jb] = None
            in_copies[jb] = pltpu.async_copy(
                table_hbm.at[pl.ds(base + j * _CHUNK, _CHUNK)],
                buf.at[jb], sins[jb])
    for b in range(_NBUF):
        if out_copies[b] is not None:
            out_copies[b].wait()


def kernel(x, pe_weight):
    del x
    return _pe_copy(pe_weight)[None]


# speedup vs baseline: 1.0757x; 1.0341x over previous
import functools

import jax
import jax.numpy as jnp
from jax import lax
from jax.experimental import pallas as pl
from jax.experimental.pallas import tpu as pltpu
from jax.experimental.pallas import tpu_sc as plsc

_ROWS = 8192
_D = 1024
_NC = 2
_NS = 16
_NW = _NC * _NS
_RPW = _ROWS // _NW
_CHUNK = 32
_NCHUNK = _RPW // _CHUNK
_NBUF = 3

_mesh = plsc.VectorSubcoreMesh(core_axis_name="c", subcore_axis_name="s")


@functools.partial(
    pl.kernel,
    out_type=jax.ShapeDtypeStruct((_ROWS, _D), jnp.float32),
    mesh=_mesh,
    scratch_types=[
        pltpu.VMEM((_NBUF, _CHUNK, _D), jnp.float32),
    ] + [pltpu.SemaphoreType.DMA] * (2 * _NBUF),
)
def _pe_copy(table_hbm, out_hbm, buf, *sems):
    sins = sems[:_NBUF]
    souts = sems[_NBUF:]
    wid = lax.axis_index("s") * _NC + lax.axis_index("c")
    base = wid * _RPW
    in_copies = [None] * _NBUF
    out_copies = [None] * _NBUF

    for i in range(min(_NBUF, _NCHUNK)):
        in_copies[i] = pltpu.async_copy(
            table_hbm.at[pl.ds(base + i * _CHUNK, _CHUNK)],
            buf.at[i], sins[i])
    for i in range(_NCHUNK):
        b = i % _NBUF
        in_copies[b].wait()
        out_copies[b] = pltpu.async_copy(
            buf.at[b], out_hbm.at[pl.ds(base + i * _CHUNK, _CHUNK)], souts[b])
        j = i + _NBUF
        if j < _NCHUNK:
            out_copies[b].wait()
            out_copies[b] = None
            in_copies[b] = pltpu.async_copy(
                table_hbm.at[pl.ds(base + j * _CHUNK, _CHUNK)],
                buf.at[b], sins[b])
    for b in range(_NBUF):
        if out_copies[b] is not None:
            out_copies[b].wait()


def kernel(x, pe_weight):
    del x
    return _pe_copy(pe_weight)[None]
